# baseline (speedup 1.0000x reference)
import jax
import jax.numpy as jnp
from jax import lax
from jax.experimental import pallas as pl
from jax.experimental.pallas import tpu as pltpu
from jax.experimental.pallas import tpu_sc as plsc

_B = 16384
_D = 64
_N = 100000
_NHA = 49920
_NHB = _N - _NHA
_NC = 2
_NS = 16
_NW = _NC * _NS
_LANES = 16
_HALF = _B // 2


def _sc_body(tensor_t_hbm, targets_hbm, centers_t_hbm, out_hbm,
             h0_v, h1_v, trow_v, idx_v, acc_v, sem_a, sem_b, sem_t, sem_i):
  wid = lax.axis_index("s") * _NC + lax.axis_index("c")
  j0 = wid
  j1 = wid + _NW

  def gather_pass(half_v, off, size, acc):
    d_idx = pltpu.async_copy(targets_hbm.at[pl.ds(0, _HALF)], idx_v, sem_i)
    for h in range(2):
      d_idx.wait()

      def step(o, a):
        g16 = idx_v[pl.ds(o * _LANES, _LANES)]
        q = g16 - off
        qc = jnp.clip(q, 0, size - 1)
        c16 = plsc.load_gather(half_v, [qc])
        t16 = trow_v[pl.ds(h * _HALF + o * _LANES, _LANES)]
        d = t16 - c16
        d = jnp.where((q >= 0) & (q < size), d, 0.0)
        return a + d * d

      acc = lax.fori_loop(0, _HALF // _LANES, step, acc)
      if h == 0:
        d_idx = pltpu.async_copy(
            targets_hbm.at[pl.ds(_HALF, _HALF)], idx_v, sem_i)
    return acc

  d_a = pltpu.async_copy(centers_t_hbm.at[j0].at[pl.ds(0, _NHA)], h0_v, sem_a)
  d_b = pltpu.async_copy(centers_t_hbm.at[j0].at[pl.ds(_NHA, _NHB)], h1_v, sem_b)
  d_t = pltpu.async_copy(tensor_t_hbm.at[j0], trow_v, sem_t)

  acc = jnp.zeros((_LANES,), jnp.float32)

  d_a.wait()
  d_t.wait()
  acc = gather_pass(h0_v, 0, _NHA, acc)

  d_a1 = pltpu.async_copy(centers_t_hbm.at[j1].at[pl.ds(0, _NHA)], h0_v, sem_a)
  d_b.wait()
  acc = gather_pass(h1_v, _NHA, _NHB, acc)

  d_b1 = pltpu.async_copy(centers_t_hbm.at[j1].at[pl.ds(_NHA, _NHB)], h1_v, sem_b)
  d_t1 = pltpu.async_copy(tensor_t_hbm.at[j1], trow_v, sem_t)
  d_a1.wait()
  d_t1.wait()
  acc = gather_pass(h0_v, 0, _NHA, acc)

  d_b1.wait()
  acc = gather_pass(h1_v, _NHA, _NHB, acc)

  acc_v[...] = acc
  pltpu.sync_copy(acc_v, out_hbm.at[wid])


@jax.jit
def kernel(tensor, targets, centers):
  targets = targets.astype(jnp.int32)
  partials = pl.kernel(
      _sc_body,
      out_type=jax.ShapeDtypeStruct((_NW, _LANES), jnp.float32),
      mesh=plsc.VectorSubcoreMesh(core_axis_name="c", subcore_axis_name="s"),
      scratch_types=[
          pltpu.VMEM((_NHA,), jnp.float32),
          pltpu.VMEM((_NHB,), jnp.float32),
          pltpu.VMEM((_B,), jnp.float32),
          pltpu.VMEM((_HALF,), jnp.int32),
          pltpu.VMEM((_LANES,), jnp.float32),
          pltpu.SemaphoreType.DMA,
          pltpu.SemaphoreType.DMA,
          pltpu.SemaphoreType.DMA,
          pltpu.SemaphoreType.DMA,
      ],
      compiler_params=pltpu.CompilerParams(needs_layout_passes=False),
  )(tensor.T, targets, centers.T)
  return 0.5 * jnp.sum(partials)


# speedup vs baseline: 1.1225x; 1.1225x over previous
import jax
import jax.numpy as jnp
from jax import lax
from jax.experimental import pallas as pl
from jax.experimental.pallas import tpu as pltpu
from jax.experimental.pallas import tpu_sc as plsc

_B = 16384
_D = 64
_N = 100000
_NHA = 49920
_NHB = _N - _NHA
_NC = 2
_NS = 16
_NW = _NC * _NS
_LANES = 16
_HALF = _B // 2
_UNROLL = 8


def _sc_body(tensor_t_hbm, targets_hbm, centers_t_hbm, out_hbm,
             h0_v, h1_v, trow_v, idx_v, acc_v, sem_a, sem_b, sem_t, sem_i):
  wid = lax.axis_index("s") * _NC + lax.axis_index("c")
  j0 = wid
  j1 = wid + _NW

  def gather_pass(half_v, off, size, acc):
    d_idx = pltpu.async_copy(targets_hbm.at[pl.ds(0, _HALF)], idx_v, sem_i)
    for h in range(2):
      d_idx.wait()

      def step(o, a):
        for u in range(_UNROLL):
          base = (o * _UNROLL + u) * _LANES
          g16 = idx_v[pl.ds(base, _LANES)]
          q = g16 - off
          qc = jnp.clip(q, 0, size - 1)
          c16 = plsc.load_gather(half_v, [qc])
          t16 = trow_v[pl.ds(h * _HALF + base, _LANES)]
          d = t16 - c16
          d = jnp.where((q >= 0) & (q < size), d, 0.0)
          a = a + d * d
        return a

      acc = lax.fori_loop(0, _HALF // (_LANES * _UNROLL), step, acc)
      if h == 0:
        d_idx = pltpu.async_copy(
            targets_hbm.at[pl.ds(_HALF, _HALF)], idx_v, sem_i)
    return acc

  d_a = pltpu.async_copy(centers_t_hbm.at[j0].at[pl.ds(0, _NHA)], h0_v, sem_a)
  d_b = pltpu.async_copy(centers_t_hbm.at[j0].at[pl.ds(_NHA, _NHB)], h1_v, sem_b)
  d_t = pltpu.async_copy(tensor_t_hbm.at[j0], trow_v, sem_t)

  acc = jnp.zeros((_LANES,), jnp.float32)

  d_a.wait()
  d_t.wait()
  acc = gather_pass(h0_v, 0, _NHA, acc)

  d_a1 = pltpu.async_copy(centers_t_hbm.at[j1].at[pl.ds(0, _NHA)], h0_v, sem_a)
  d_b.wait()
  acc = gather_pass(h1_v, _NHA, _NHB, acc)

  d_b1 = pltpu.async_copy(centers_t_hbm.at[j1].at[pl.ds(_NHA, _NHB)], h1_v, sem_b)
  d_t1 = pltpu.async_copy(tensor_t_hbm.at[j1], trow_v, sem_t)
  d_a1.wait()
  d_t1.wait()
  acc = gather_pass(h0_v, 0, _NHA, acc)

  d_b1.wait()
  acc = gather_pass(h1_v, _NHA, _NHB, acc)

  acc_v[...] = acc
  pltpu.sync_copy(acc_v, out_hbm.at[wid])


@jax.jit
def kernel(tensor, targets, centers):
  targets = targets.astype(jnp.int32)
  partials = pl.kernel(
      _sc_body,
      out_type=jax.ShapeDtypeStruct((_NW, _LANES), jnp.float32),
      mesh=plsc.VectorSubcoreMesh(core_axis_name="c", subcore_axis_name="s"),
      scratch_types=[
          pltpu.VMEM((_NHA,), jnp.float32),
          pltpu.VMEM((_NHB,), jnp.float32),
          pltpu.VMEM((_B,), jnp.float32),
          pltpu.VMEM((_HALF,), jnp.int32),
          pltpu.VMEM((_LANES,), jnp.float32),
          pltpu.SemaphoreType.DMA,
          pltpu.SemaphoreType.DMA,
          pltpu.SemaphoreType.DMA,
          pltpu.SemaphoreType.DMA,
      ],
      compiler_params=pltpu.CompilerParams(needs_layout_passes=False),
  )(tensor.T, targets, centers.T)
  return 0.5 * jnp.sum(partials)


# speedup vs baseline: 1.3718x; 1.2221x over previous
import jax
import jax.numpy as jnp
from jax import lax
from jax.experimental import pallas as pl
from jax.experimental.pallas import tpu as pltpu
from jax.experimental.pallas import tpu_sc as plsc

_B = 16384
_D = 64
_N = 100000
_NC = 2
_NS = 16
_NW = _NC * _NS
_LANES = 16
_HALF = _B // 2
_UNROLL = 8


def _sc_body(tensor_t_hbm, targets_hbm, centers_t_hbm, out_hbm,
             plane_v, trow_v, idx_v, acc_v, psem, tsem, isem):
  wid = lax.axis_index("s") * _NC + lax.axis_index("c")

  acc = jnp.zeros((_LANES,), jnp.float32)
  for p in range(2):
    j = wid + _NW * p
    d_plane = pltpu.async_copy(centers_t_hbm.at[j], plane_v, psem)
    d_trow = pltpu.async_copy(tensor_t_hbm.at[j], trow_v, tsem)
    d_idx = pltpu.async_copy(targets_hbm.at[pl.ds(0, _HALF)], idx_v, isem)
    d_plane.wait()
    d_trow.wait()

    for h in range(2):
      d_idx.wait()

      def step(o, a):
        for u in range(_UNROLL):
          base = (o * _UNROLL + u) * _LANES
          g16 = idx_v[pl.ds(base, _LANES)]
          c16 = plsc.load_gather(plane_v, [g16])
          t16 = trow_v[pl.ds(h * _HALF + base, _LANES)]
          d = t16 - c16
          a = a + d * d
        return a

      acc = lax.fori_loop(0, _HALF // (_LANES * _UNROLL), step, acc)
      if h == 0:
        d_idx = pltpu.async_copy(
            targets_hbm.at[pl.ds(_HALF, _HALF)], idx_v, isem)

  acc_v[...] = acc
  pltpu.sync_copy(acc_v, out_hbm.at[wid])


@jax.jit
def kernel(tensor, targets, centers):
  targets = targets.astype(jnp.int32)
  partials = pl.kernel(
      _sc_body,
      out_type=jax.ShapeDtypeStruct((_NW, _LANES), jnp.float32),
      mesh=plsc.VectorSubcoreMesh(core_axis_name="c", subcore_axis_name="s"),
      scratch_types=[
          pltpu.VMEM((_N,), jnp.float32),
          pltpu.VMEM((_B,), jnp.float32),
          pltpu.VMEM((_HALF,), jnp.int32),
          pltpu.VMEM((_LANES,), jnp.float32),
          pltpu.SemaphoreType.DMA,
          pltpu.SemaphoreType.DMA,
          pltpu.SemaphoreType.DMA,
      ],
      compiler_params=pltpu.CompilerParams(needs_layout_passes=False),
  )(tensor.T, targets, centers.T)
  return 0.5 * jnp.sum(partials)
